# baseline (speedup 1.0000x reference)
import dataclasses
import functools

import jax
import jax.numpy as jnp
from jax import lax
from jax.experimental import pallas as pl
from jax.experimental.pallas import tpu as pltpu
from jax.experimental.pallas import tpu_sc as plsc

N = 100000
D = 256
W = 160
NCHUNK = N // W
NW = 32
KMAX = -(-NCHUNK // NW)
L = 16


def _sc_gather(table, xflat):
    mesh = plsc.VectorSubcoreMesh(core_axis_name="c", subcore_axis_name="s")
    cp = pltpu.CompilerParams()
    if "needs_layout_passes" in pltpu.CompilerParams.__dataclass_fields__:
        cp = dataclasses.replace(cp, needs_layout_passes=False)

    @functools.partial(
        pl.kernel,
        mesh=mesh,
        compiler_params=cp,
        out_type=jax.ShapeDtypeStruct((N, D), jnp.float32),
        scratch_types=[
            pltpu.VMEM((2 * W,), jnp.int32),
            pltpu.VMEM((2, W // 2), jnp.int32),
            pltpu.VMEM((W, D), jnp.float32),
            pltpu.SemaphoreType.DMA,
        ],
    )
    def k(table_hbm, xflat_hbm, out_hbm, xv, idxv, rows, sem):
        wid = lax.axis_index("s") * 2 + lax.axis_index("c")

        @pl.loop(0, KMAX)
        def _(kk):
            chunk = kk * NW + wid

            @pl.when(chunk < NCHUNK)
            def _():
                pltpu.sync_copy(xflat_hbm.at[pl.ds(chunk * 2 * W, 2 * W)], xv)

                iota = lax.iota(jnp.int32, L)
                for g in range(W // L):
                    ev = plsc.load_gather(xv, [iota * 2 + (2 * L * g)])
                    od = plsc.load_gather(xv, [iota * 2 + (2 * L * g + 1)])
                    idxv[g // 5, pl.ds((g % 5) * L, L)] = ev * 3 + od

                c0 = pltpu.async_copy(
                    table_hbm.at[idxv.at[0]], rows.at[pl.ds(0, W // 2)], sem)
                c1 = pltpu.async_copy(
                    table_hbm.at[idxv.at[1]], rows.at[pl.ds(W // 2, W // 2)], sem)
                c0.wait()
                c1.wait()

                pltpu.sync_copy(rows, out_hbm.at[pl.ds(chunk * W, W)])

    return k(table, xflat)


def kernel(x, node_type_table, num_inv_pred_table):
    combined = jnp.concatenate(
        [jnp.repeat(node_type_table, 3, axis=0),
         jnp.tile(num_inv_pred_table, (3, 1))],
        axis=1,
    )
    xflat = x.astype(jnp.int32).reshape(-1)
    return _sc_gather(combined, xflat)
